# baseline (speedup 1.0000x reference)
import jax
import jax.numpy as jnp
from jax.experimental import pallas as pl

_B = 1024
_HID_CNN = 32
_NUM_OBJ = 5
_HID = 512
_EMB = 32
_MLP_IN = 25
_NP = 25
_L1 = 160
_LH = 800

_BB1 = 128
_G1 = _B // _BB1
_BB2 = 256
_G2 = _B // _BB2

_EPS = 1e-5


def _c1_body(x_ref, m_ref, b1_ref, h_ref, st_ref):
    xf = x_ref[...].reshape(_BB1, 7500)
    hs = []
    for i in range(5):
        acc = None
        for c in range(3):
            k = c * 5 + i
            x = xf[:, k * 500:(k + 1) * 500]
            p = jnp.dot(x, m_ref[c], preferred_element_type=jnp.float32)
            acc = p if acc is None else acc + p
        hs.append(acc + b1_ref[...])
    h = jnp.concatenate(hs, axis=1)
    h_ref[...] = h
    st = jnp.concatenate(
        [jnp.sum(h, axis=0, keepdims=True),
         jnp.sum(h * h, axis=0, keepdims=True)], axis=0)

    @pl.when(pl.program_id(0) == 0)
    def _():
        st_ref[...] = st

    @pl.when(pl.program_id(0) > 0)
    def _():
        st_ref[...] += st


def _c2_body(h_ref, st_ref, fold_ref, spread_ref, g_ref, bt_ref, w2_ref,
             b2_ref, f1_ref, f1b_ref, f2_ref, f2b_ref, lg_ref, lb_ref,
             f3_ref, f3b_ref, o_ref):
    n = jnp.float32(_B * _NP)
    stf = jnp.dot(st_ref[...], fold_ref[...],
                  preferred_element_type=jnp.float32)
    mean = stf[0:1, :] / n
    var = stf[1:2, :] / n - mean * mean
    sc32 = g_ref[...] * jax.lax.rsqrt(var + _EPS)
    sh32 = bt_ref[...] - mean * sc32
    sc = jnp.dot(sc32, spread_ref[...],
                 preferred_element_type=jnp.float32)
    sh = jnp.dot(sh32, spread_ref[...],
                 preferred_element_type=jnp.float32)

    r = jnp.maximum(h_ref[...] * sc + sh, 0.0)
    s = jnp.dot(r, w2_ref[...], preferred_element_type=jnp.float32)
    s = jax.nn.sigmoid(s + b2_ref[...])

    for o2 in range(_NUM_OBJ):
        x = s[:, o2 * _MLP_IN:(o2 + 1) * _MLP_IN]
        x = jnp.dot(x, f1_ref[...], preferred_element_type=jnp.float32)
        x = jnp.maximum(x + f1b_ref[...], 0.0)
        x = jnp.dot(x, f2_ref[...], preferred_element_type=jnp.float32)
        x = x + f2b_ref[...]

        mu = jnp.mean(x, axis=-1, keepdims=True)
        d = x - mu
        v = jnp.mean(d * d, axis=-1, keepdims=True)
        x = d * jax.lax.rsqrt(v + _EPS) * lg_ref[...] + lb_ref[...]
        x = jnp.maximum(x, 0.0)

        o = jnp.dot(x, f3_ref[...], preferred_element_type=jnp.float32)
        o_ref[o2, :, :] = o + f3b_ref[...]


def kernel(obs, cnn1_w, cnn1_b, bn1_g, bn1_b, cnn2_w, cnn2_b,
           fc1_w, fc1_b, fc2_w, fc2_b, ln_g, ln_b, fc3_w, fc3_b):

    w4 = cnn1_w.transpose(1, 2, 3, 0)
    eye5 = jnp.eye(5, dtype=obs.dtype)
    m6 = w4[:, :, None, :, None, :] * eye5[None, None, :, None, :, None]
    m = m6.reshape(3, 500, _L1)
    b1 = jnp.tile(cnn1_b, _NP // 5).reshape(1, _L1)

    w2m = cnn2_w.reshape(_NUM_OBJ, _HID_CNN)
    eye25 = jnp.eye(_NP, dtype=obs.dtype)
    w2big = (eye25[:, None, None, :] * w2m.T[None, :, :, None]
             ).reshape(_LH, _NUM_OBJ * _NP)
    b2 = jnp.repeat(cnn2_b, _NP).reshape(1, _NUM_OBJ * _NP)

    f1 = fc1_w.T
    f2 = fc2_w.T
    f3 = fc3_w.T

    fold = jnp.tile(jnp.eye(_HID_CNN, dtype=obs.dtype), (_NP, 1))
    spread = fold.T

    h_pre, st = pl.pallas_call(
        _c1_body,
        grid=(_G1,),
        in_specs=[
            pl.BlockSpec((_BB1, 3, 50, 50), lambda i: (i, 0, 0, 0)),
            pl.BlockSpec((3, 500, _L1), lambda i: (0, 0, 0)),
            pl.BlockSpec((1, _L1), lambda i: (0, 0)),
        ],
        out_specs=[
            pl.BlockSpec((_BB1, _LH), lambda i: (i, 0)),
            pl.BlockSpec((2, _LH), lambda i: (0, 0)),
        ],
        out_shape=[
            jax.ShapeDtypeStruct((_B, _LH), jnp.float32),
            jax.ShapeDtypeStruct((2, _LH), jnp.float32),
        ],
    )(obs, m, b1)

    rep = lambda i: (0, 0)
    out = pl.pallas_call(
        _c2_body,
        grid=(_G2,),
        in_specs=[
            pl.BlockSpec((_BB2, _LH), lambda i: (i, 0)),
            pl.BlockSpec((2, _LH), rep),
            pl.BlockSpec((_LH, _HID_CNN), rep),
            pl.BlockSpec((_HID_CNN, _LH), rep),
            pl.BlockSpec((1, _HID_CNN), rep),
            pl.BlockSpec((1, _HID_CNN), rep),
            pl.BlockSpec((_LH, _NUM_OBJ * _NP), rep),
            pl.BlockSpec((1, _NUM_OBJ * _NP), rep),
            pl.BlockSpec((_MLP_IN, _HID), rep),
            pl.BlockSpec((1, _HID), rep),
            pl.BlockSpec((_HID, _HID), rep),
            pl.BlockSpec((1, _HID), rep),
            pl.BlockSpec((1, _HID), rep),
            pl.BlockSpec((1, _HID), rep),
            pl.BlockSpec((_HID, _EMB), rep),
            pl.BlockSpec((1, _EMB), rep),
        ],
        out_specs=pl.BlockSpec((_NUM_OBJ, _BB2, _EMB), lambda i: (0, i, 0)),
        out_shape=jax.ShapeDtypeStruct((_NUM_OBJ, _B, _EMB), jnp.float32),
    )(h_pre, st, fold, spread, bn1_g.reshape(1, -1), bn1_b.reshape(1, -1),
      w2big, b2, f1, fc1_b.reshape(1, -1), f2, fc2_b.reshape(1, -1),
      ln_g.reshape(1, -1), ln_b.reshape(1, -1), f3, fc3_b.reshape(1, -1))

    return out.transpose(1, 0, 2)


# speedup vs baseline: 1.0007x; 1.0007x over previous
import jax
import jax.numpy as jnp
from jax.experimental import pallas as pl
from jax.experimental.pallas import tpu as pltpu
from jax._src.pallas.mosaic.primitives import make_async_copy as _mk_copy

_B = 1024
_HID_CNN = 32
_NUM_OBJ = 5
_HID = 512
_EMB = 32
_MLP_IN = 25
_NP = 25
_L1 = 160
_LH = 800

_BB1 = 128
_G1 = _B // _BB1
_BB2 = 256
_G2 = _B // _BB2

_EPS = 1e-5


def _c1_body(x_hbm, m_ref, b1_ref, h_ref, st_ref, buf, sem):
    g = pl.program_id(0)
    slot = jax.lax.rem(g, 2)
    nxt = jax.lax.rem(g + 1, 2)

    @pl.when(g == 0)
    def _():
        _mk_copy(x_hbm.at[pl.ds(0, _BB1)], buf.at[0],
                              sem.at[0]).start()

    @pl.when(g + 1 < _G1)
    def _():
        _mk_copy(x_hbm.at[pl.ds((g + 1) * _BB1, _BB1)],
                              buf.at[nxt], sem.at[nxt]).start()

    _mk_copy(x_hbm.at[pl.ds(g * _BB1, _BB1)], buf.at[slot],
                          sem.at[slot]).wait()

    xf = buf[slot].reshape(_BB1, 7500)
    hs = []
    for i in range(5):
        acc = None
        for c in range(3):
            k = c * 5 + i
            x = xf[:, k * 500:(k + 1) * 500]
            p = jnp.dot(x, m_ref[c], preferred_element_type=jnp.float32)
            acc = p if acc is None else acc + p
        hs.append(acc + b1_ref[...])
    h = jnp.concatenate(hs, axis=1)
    h_ref[...] = h
    st = jnp.concatenate(
        [jnp.sum(h, axis=0, keepdims=True),
         jnp.sum(h * h, axis=0, keepdims=True)], axis=0)

    @pl.when(pl.program_id(0) == 0)
    def _():
        st_ref[...] = st

    @pl.when(pl.program_id(0) > 0)
    def _():
        st_ref[...] += st


def _c2_body(h_ref, st_ref, fold_ref, spread_ref, g_ref, bt_ref, w2_ref,
             b2_ref, f1_ref, f1b_ref, f2_ref, f2b_ref, lg_ref, lb_ref,
             f3_ref, f3b_ref, o_ref):
    n = jnp.float32(_B * _NP)
    stf = jnp.dot(st_ref[...], fold_ref[...],
                  preferred_element_type=jnp.float32)
    mean = stf[0:1, :] / n
    var = stf[1:2, :] / n - mean * mean
    sc32 = g_ref[...] * jax.lax.rsqrt(var + _EPS)
    sh32 = bt_ref[...] - mean * sc32
    sc = jnp.dot(sc32, spread_ref[...],
                 preferred_element_type=jnp.float32)
    sh = jnp.dot(sh32, spread_ref[...],
                 preferred_element_type=jnp.float32)

    r = jnp.maximum(h_ref[...] * sc + sh, 0.0)
    s = jnp.dot(r, w2_ref[...], preferred_element_type=jnp.float32)
    s = jax.nn.sigmoid(s + b2_ref[...])

    for o2 in range(_NUM_OBJ):
        x = s[:, o2 * _MLP_IN:(o2 + 1) * _MLP_IN]
        x = jnp.dot(x, f1_ref[...], preferred_element_type=jnp.float32)
        x = jnp.maximum(x + f1b_ref[...], 0.0)
        x = jnp.dot(x, f2_ref[...], preferred_element_type=jnp.float32)
        x = x + f2b_ref[...]

        mu = jnp.mean(x, axis=-1, keepdims=True)
        d = x - mu
        v = jnp.mean(d * d, axis=-1, keepdims=True)
        x = d * jax.lax.rsqrt(v + _EPS) * lg_ref[...] + lb_ref[...]
        x = jnp.maximum(x, 0.0)

        o = jnp.dot(x, f3_ref[...], preferred_element_type=jnp.float32)
        o_ref[o2, :, :] = o + f3b_ref[...]


def kernel(obs, cnn1_w, cnn1_b, bn1_g, bn1_b, cnn2_w, cnn2_b,
           fc1_w, fc1_b, fc2_w, fc2_b, ln_g, ln_b, fc3_w, fc3_b):

    w4 = cnn1_w.transpose(1, 2, 3, 0)
    eye5 = jnp.eye(5, dtype=obs.dtype)
    m6 = w4[:, :, None, :, None, :] * eye5[None, None, :, None, :, None]
    m = m6.reshape(3, 500, _L1)
    b1 = jnp.tile(cnn1_b, _NP // 5).reshape(1, _L1)

    w2m = cnn2_w.reshape(_NUM_OBJ, _HID_CNN)
    eye25 = jnp.eye(_NP, dtype=obs.dtype)
    w2big = (eye25[:, None, None, :] * w2m.T[None, :, :, None]
             ).reshape(_LH, _NUM_OBJ * _NP)
    b2 = jnp.repeat(cnn2_b, _NP).reshape(1, _NUM_OBJ * _NP)

    f1 = fc1_w.T
    f2 = fc2_w.T
    f3 = fc3_w.T

    fold = jnp.tile(jnp.eye(_HID_CNN, dtype=obs.dtype), (_NP, 1))
    spread = fold.T

    h_pre, st = pl.pallas_call(
        _c1_body,
        grid=(_G1,),
        in_specs=[
            pl.BlockSpec(memory_space=pltpu.MemorySpace.HBM),
            pl.BlockSpec((3, 500, _L1), lambda i: (0, 0, 0)),
            pl.BlockSpec((1, _L1), lambda i: (0, 0)),
        ],
        out_specs=[
            pl.BlockSpec((_BB1, _LH), lambda i: (i, 0)),
            pl.BlockSpec((2, _LH), lambda i: (0, 0)),
        ],
        out_shape=[
            jax.ShapeDtypeStruct((_B, _LH), jnp.float32),
            jax.ShapeDtypeStruct((2, _LH), jnp.float32),
        ],
        scratch_shapes=[
            pltpu.VMEM((2, _BB1, 3, 50, 50), jnp.float32),
            pltpu.SemaphoreType.DMA((2,)),
        ],
    )(obs, m, b1)

    rep = lambda i: (0, 0)
    out = pl.pallas_call(
        _c2_body,
        grid=(_G2,),
        in_specs=[
            pl.BlockSpec((_BB2, _LH), lambda i: (i, 0)),
            pl.BlockSpec((2, _LH), rep),
            pl.BlockSpec((_LH, _HID_CNN), rep),
            pl.BlockSpec((_HID_CNN, _LH), rep),
            pl.BlockSpec((1, _HID_CNN), rep),
            pl.BlockSpec((1, _HID_CNN), rep),
            pl.BlockSpec((_LH, _NUM_OBJ * _NP), rep),
            pl.BlockSpec((1, _NUM_OBJ * _NP), rep),
            pl.BlockSpec((_MLP_IN, _HID), rep),
            pl.BlockSpec((1, _HID), rep),
            pl.BlockSpec((_HID, _HID), rep),
            pl.BlockSpec((1, _HID), rep),
            pl.BlockSpec((1, _HID), rep),
            pl.BlockSpec((1, _HID), rep),
            pl.BlockSpec((_HID, _EMB), rep),
            pl.BlockSpec((1, _EMB), rep),
        ],
        out_specs=pl.BlockSpec((_NUM_OBJ, _BB2, _EMB), lambda i: (0, i, 0)),
        out_shape=jax.ShapeDtypeStruct((_NUM_OBJ, _B, _EMB), jnp.float32),
    )(h_pre, st, fold, spread, bn1_g.reshape(1, -1), bn1_b.reshape(1, -1),
      w2big, b2, f1, fc1_b.reshape(1, -1), f2, fc2_b.reshape(1, -1),
      ln_g.reshape(1, -1), ln_b.reshape(1, -1), f3, fc3_b.reshape(1, -1))

    return out.transpose(1, 0, 2)


# speedup vs baseline: 1.2624x; 1.2615x over previous
import jax
import jax.numpy as jnp
from jax.experimental import pallas as pl
from jax.experimental.pallas import tpu as pltpu
from jax._src.pallas.mosaic.primitives import make_async_copy as _mk_copy

_B = 1024
_HID_CNN = 32
_NUM_OBJ = 5
_HID = 512
_EMB = 32
_MLP_IN = 25
_NP = 25
_L1 = 160
_LH = 800

_BB1 = 256
_G1 = _B // _BB1
_BB2 = 256
_G2 = _B // _BB2

_EPS = 1e-5


def _c1_body(x_ref, m_ref, b1_ref, h_ref, st_ref):
    hs = []
    for i in range(5):
        acc = None
        for c in range(3):
            k = c * 5 + i
            x = x_ref[:, k * 500:(k + 1) * 500]
            p = jnp.dot(x, m_ref[c], preferred_element_type=jnp.float32)
            acc = p if acc is None else acc + p
        hs.append(acc + b1_ref[...])
    h = jnp.concatenate(hs, axis=1)
    h_ref[...] = h
    st = jnp.concatenate(
        [jnp.sum(h, axis=0, keepdims=True),
         jnp.sum(h * h, axis=0, keepdims=True)], axis=0)

    @pl.when(pl.program_id(0) == 0)
    def _():
        st_ref[...] = st

    @pl.when(pl.program_id(0) > 0)
    def _():
        st_ref[...] += st


def _c2_body(h_ref, st_ref, fold_ref, spread_ref, g_ref, bt_ref, w2_ref,
             b2_ref, f1_ref, f1b_ref, f2_ref, f2b_ref, lg_ref, lb_ref,
             f3_ref, f3b_ref, o_ref):
    n = jnp.float32(_B * _NP)
    stf = jnp.dot(st_ref[...], fold_ref[...],
                  preferred_element_type=jnp.float32)
    mean = stf[0:1, :] / n
    var = stf[1:2, :] / n - mean * mean
    sc32 = g_ref[...] * jax.lax.rsqrt(var + _EPS)
    sh32 = bt_ref[...] - mean * sc32
    sc = jnp.dot(sc32, spread_ref[...],
                 preferred_element_type=jnp.float32)
    sh = jnp.dot(sh32, spread_ref[...],
                 preferred_element_type=jnp.float32)

    r = jnp.maximum(h_ref[...] * sc + sh, 0.0)
    s = jnp.dot(r, w2_ref[...], preferred_element_type=jnp.float32)
    s = jax.nn.sigmoid(s + b2_ref[...])

    for o2 in range(_NUM_OBJ):
        x = s[:, o2 * _MLP_IN:(o2 + 1) * _MLP_IN]
        x = jnp.dot(x, f1_ref[...], preferred_element_type=jnp.float32)
        x = jnp.maximum(x + f1b_ref[...], 0.0)
        x = jnp.dot(x, f2_ref[...], preferred_element_type=jnp.float32)
        x = x + f2b_ref[...]

        mu = jnp.mean(x, axis=-1, keepdims=True)
        d = x - mu
        v = jnp.mean(d * d, axis=-1, keepdims=True)
        x = d * jax.lax.rsqrt(v + _EPS) * lg_ref[...] + lb_ref[...]
        x = jnp.maximum(x, 0.0)

        o = jnp.dot(x, f3_ref[...], preferred_element_type=jnp.float32)
        o_ref[o2, :, :] = o + f3b_ref[...]


def kernel(obs, cnn1_w, cnn1_b, bn1_g, bn1_b, cnn2_w, cnn2_b,
           fc1_w, fc1_b, fc2_w, fc2_b, ln_g, ln_b, fc3_w, fc3_b):

    w4 = cnn1_w.transpose(1, 2, 3, 0)
    eye5 = jnp.eye(5, dtype=obs.dtype)
    m6 = w4[:, :, None, :, None, :] * eye5[None, None, :, None, :, None]
    m = m6.reshape(3, 500, _L1)
    b1 = jnp.tile(cnn1_b, _NP // 5).reshape(1, _L1)

    w2m = cnn2_w.reshape(_NUM_OBJ, _HID_CNN)
    eye25 = jnp.eye(_NP, dtype=obs.dtype)
    w2big = (eye25[:, None, None, :] * w2m.T[None, :, :, None]
             ).reshape(_LH, _NUM_OBJ * _NP)
    b2 = jnp.repeat(cnn2_b, _NP).reshape(1, _NUM_OBJ * _NP)

    f1 = fc1_w.T
    f2 = fc2_w.T
    f3 = fc3_w.T

    fold = jnp.tile(jnp.eye(_HID_CNN, dtype=obs.dtype), (_NP, 1))
    spread = fold.T

    h_pre, st = pl.pallas_call(
        _c1_body,
        grid=(_G1,),
        in_specs=[
            pl.BlockSpec((_BB1, 7500), lambda i: (i, 0)),
            pl.BlockSpec((3, 500, _L1), lambda i: (0, 0, 0)),
            pl.BlockSpec((1, _L1), lambda i: (0, 0)),
        ],
        out_specs=[
            pl.BlockSpec((_BB1, _LH), lambda i: (i, 0)),
            pl.BlockSpec((2, _LH), lambda i: (0, 0)),
        ],
        out_shape=[
            jax.ShapeDtypeStruct((_B, _LH), jnp.float32),
            jax.ShapeDtypeStruct((2, _LH), jnp.float32),
        ],
    )(obs.astype(jnp.bfloat16).reshape(_B, 7500), m.astype(jnp.bfloat16), b1)

    rep = lambda i: (0, 0)
    out = pl.pallas_call(
        _c2_body,
        grid=(_G2,),
        in_specs=[
            pl.BlockSpec((_BB2, _LH), lambda i: (i, 0)),
            pl.BlockSpec((2, _LH), rep),
            pl.BlockSpec((_LH, _HID_CNN), rep),
            pl.BlockSpec((_HID_CNN, _LH), rep),
            pl.BlockSpec((1, _HID_CNN), rep),
            pl.BlockSpec((1, _HID_CNN), rep),
            pl.BlockSpec((_LH, _NUM_OBJ * _NP), rep),
            pl.BlockSpec((1, _NUM_OBJ * _NP), rep),
            pl.BlockSpec((_MLP_IN, _HID), rep),
            pl.BlockSpec((1, _HID), rep),
            pl.BlockSpec((_HID, _HID), rep),
            pl.BlockSpec((1, _HID), rep),
            pl.BlockSpec((1, _HID), rep),
            pl.BlockSpec((1, _HID), rep),
            pl.BlockSpec((_HID, _EMB), rep),
            pl.BlockSpec((1, _EMB), rep),
        ],
        out_specs=pl.BlockSpec((_NUM_OBJ, _BB2, _EMB), lambda i: (0, i, 0)),
        out_shape=jax.ShapeDtypeStruct((_NUM_OBJ, _B, _EMB), jnp.float32),
    )(h_pre, st, fold, spread, bn1_g.reshape(1, -1), bn1_b.reshape(1, -1),
      w2big, b2, f1, fc1_b.reshape(1, -1), f2, fc2_b.reshape(1, -1),
      ln_g.reshape(1, -1), ln_b.reshape(1, -1), f3, fc3_b.reshape(1, -1))

    return out.transpose(1, 0, 2)
